# baseline (speedup 1.0000x reference)
import functools

import jax
import jax.numpy as jnp
from jax import lax
from jax.experimental import pallas as pl
from jax.experimental.pallas import tpu as pltpu
from jax.experimental.pallas import tpu_sc as plsc

VOCAB = 1000000
DIM = 64
SEQ = 200
BATCH = 4096

NC = 2
NS = 16
NW = NC * NS
G = BATCH // NW
K = 4
STEPS = SEQ // (2 * K)

_mesh = plsc.VectorSubcoreMesh(core_axis_name="c", subcore_axis_name="s")


@functools.partial(
    pl.kernel,
    mesh=_mesh,
    out_type=jax.ShapeDtypeStruct((SEQ, BATCH, DIM), jnp.float32),
    scratch_types=[
        pltpu.VMEM((SEQ, G), jnp.int32),
        pltpu.VMEM((K, G, DIM), jnp.float32),
        pltpu.VMEM((K, G, DIM), jnp.float32),
        pltpu.SemaphoreType.DMA,
        pltpu.SemaphoreType.DMA,
        pltpu.SemaphoreType.DMA,
    ],
    compiler_params=pltpu.CompilerParams(use_tc_tiling_on_sc=False),
)
def _sc_gather(table_hbm, idx_hbm, out_hbm, idx_v, rows0, rows1, gsem, wsem0, wsem1):
    wid = lax.axis_index("s") * NC + lax.axis_index("c")
    wb = wid * G
    pltpu.sync_copy(idx_hbm.at[:, pl.ds(wb, G)], idx_v)

    def fire_and_drain(t, rows_v):
        handles = [
            pltpu.async_copy(
                table_hbm.at[idx_v.at[t * K + k]],
                rows_v.at[k],
                gsem,
            )
            for k in range(K)
        ]
        for h in handles:
            h.wait()

    def start_write(t, rows_v, wsem):
        pltpu.async_copy(
            rows_v, out_hbm.at[pl.ds(t * K, K), pl.ds(wb, G), :], wsem
        )

    def wait_write(rows_v, wsem):
        pltpu.make_async_copy(
            rows_v, out_hbm.at[pl.ds(0, K), pl.ds(wb, G), :], wsem
        ).wait()

    fire_and_drain(0, rows0)
    start_write(0, rows0, wsem0)
    fire_and_drain(1, rows1)
    start_write(1, rows1, wsem1)

    def body(c, carry):
        t0 = 2 * c
        wait_write(rows0, wsem0)
        fire_and_drain(t0, rows0)
        start_write(t0, rows0, wsem0)
        wait_write(rows1, wsem1)
        fire_and_drain(t0 + 1, rows1)
        start_write(t0 + 1, rows1, wsem1)
        return carry

    lax.fori_loop(1, STEPS, body, 0)

    wait_write(rows0, wsem0)
    wait_write(rows1, wsem1)


def kernel(src_input, word_lut):
    idx = src_input.reshape(SEQ, BATCH)
    return _sc_gather(word_lut, idx)
